# baseline (speedup 1.0000x reference)
import functools

import jax
import jax.numpy as jnp
from jax import lax
from jax.experimental import pallas as pl
from jax.experimental.pallas import tpu as pltpu
from jax.experimental.pallas import tpu_sc as plsc

BATCH = 16384
HIST = 200
D = 64
VOCAB = 2048
L = 16
NDG = 4
NBG = 8
DG = D // NDG
BG = BATCH // NBG
HB = 8
NHB = HIST // HB
NIV = BG // L
TILE_W = 1024
XBLK_W = HB * BG
STG_W = DG * BG
HALF_STG = STG_W // 2

_mesh = plsc.VectorSubcoreMesh(core_axis_name="c", subcore_axis_name="s")


@functools.partial(
    pl.kernel,
    mesh=_mesh,
    compiler_params=pltpu.CompilerParams(
        use_tc_tiling_on_sc=False, needs_layout_passes=False),
    out_type=jax.ShapeDtypeStruct((HIST * D * BATCH,), jnp.float32),
    scratch_types=[
        pltpu.VMEM((DG * VOCAB,), jnp.float32),
        pltpu.VMEM((XBLK_W,), jnp.int32),
        pltpu.VMEM((STG_W,), jnp.float32),
        pltpu.VMEM((STG_W,), jnp.float32),
        pltpu.SemaphoreType.DMA,
        pltpu.SemaphoreType.DMA,
    ],
)
def _lookup_kernel(x4_hbm, tt_hbm, out_hbm, tbl_v, xblk_v, stg_a, stg_b,
                   sem_a, sem_b):
    wid = lax.axis_index("s") * 2 + lax.axis_index("c")
    dg = wid % NDG
    bg = wid // NDG
    d0 = dg * DG
    b0 = bg * BG

    pltpu.sync_copy(tt_hbm.at[pl.ds(d0 * VOCAB, DG * VOCAB)], tbl_v)

    def store_descs(h, stg, sem):
        return [
            pltpu.make_async_copy(
                stg.at[pl.ds(j * HALF_STG, HALF_STG)],
                out_hbm.at[pl.ds(
                    ((h * 8 + 2 * dg + j) * 128 + 16 * bg) * TILE_W,
                    HALF_STG)],
                sem)
            for j in range(2)
        ]

    def hblock(hb, carry):
        pltpu.sync_copy(
            x4_hbm.at[pl.ds((hb * 128 + 16 * bg) * TILE_W, XBLK_W)], xblk_v)
        for hl in range(HB):
            h = hb * HB + hl
            stg, sem = (stg_a, sem_a) if hl % 2 == 0 else (stg_b, sem_b)

            def drain():
                for d_ in store_descs(h, stg, sem):
                    d_.wait()

            if hl >= 2:
                drain()
            else:
                @pl.when(hb > 0)
                def _():
                    drain()

            @plsc.parallel_loop(0, NIV, unroll=8)
            def inner(i):
                base = (i // 8) * TILE_W + (i % 8) * L
                xv = xblk_v[pl.ds(base + hl * 128, L)]
                for dl in range(DG):
                    off = base + (dl // 8) * HALF_STG + (dl % 8) * 128
                    tbl_slice = tbl_v.at[pl.ds(dl * VOCAB, VOCAB)]
                    stg[pl.ds(off, L)] = plsc.load_gather(tbl_slice, [xv])

            for d_ in store_descs(h, stg, sem):
                d_.start()
        return carry

    lax.fori_loop(0, NHB, hblock, 0)

    for d_ in store_descs(HIST - 2, stg_a, sem_a):
        d_.wait()
    for d_ in store_descs(HIST - 1, stg_b, sem_b):
        d_.wait()


def kernel(x, table):
    x4 = (x.astype(jnp.int32)
          .reshape(128, 128, NHB, HB)
          .transpose(2, 0, 3, 1)
          .reshape(-1))
    tt = table.T.reshape(-1)
    out_f = _lookup_kernel(x4, tt)
    out5 = out_f.reshape(HIST, 8, 128, 8, 128)
    return out5.transpose(2, 4, 0, 1, 3).reshape(BATCH, HIST, D)
